# Initial kernel scaffold; baseline (speedup 1.0000x reference)
import jax
import jax.numpy as jnp
from jax.experimental import pallas as pl


def kernel(h_prob, edge_weight, edge_index, edge_type, r_index):
    raise NotImplementedError("write your pallas kernel here")


# speedup vs baseline: 3.4832x; 3.4832x over previous
import functools

import jax
import jax.numpy as jnp
from jax import lax
from jax.experimental import pallas as pl
from jax.experimental.pallas import tpu as pltpu
from jax.experimental.pallas import tpu_sc as plsc

L = 16
NC = 2
NS = 16
NPART = 4
CHUNK = 1600
GW = 128
STG = 1664
COMB = 10000


def _build(B, N, E):
    e_per_tile = E // NPART
    nchunk = e_per_tile // CHUNK
    assert e_per_tile % CHUNK == 0 and nchunk % 2 == 0
    assert N % COMB == 0 and COMB % L == 0 and CHUNK % L == 0

    mesh = plsc.VectorSubcoreMesh(
        core_axis_name="c", subcore_axis_name="s",
        num_cores=NC, num_subcores=NS)

    @functools.partial(
        pl.kernel,
        out_type=jax.ShapeDtypeStruct((B * N,), jnp.float32),
        mesh=mesh,
        compiler_params=pltpu.CompilerParams(needs_layout_passes=False),
        scratch_types=[
            pltpu.VMEM((N,), jnp.float32),
            pltpu.VMEM((CHUNK,), jnp.int32),
            pltpu.VMEM((CHUNK,), jnp.int32),
            pltpu.VMEM((CHUNK,), jnp.int32),
            pltpu.VMEM((CHUNK,), jnp.int32),
            pltpu.VMEM((CHUNK,), jnp.int32),
            pltpu.VMEM((CHUNK,), jnp.int32),
            pltpu.VMEM((CHUNK,), jnp.float32),
            pltpu.VMEM((CHUNK,), jnp.float32),
            pltpu.VMEM((STG,), jnp.int32),
            pltpu.VMEM((STG,), jnp.int32),
            pltpu.VMEM((STG,), jnp.float32),
            pltpu.VMEM((STG,), jnp.float32),
            pltpu.VMEM((COMB,), jnp.float32),
            pltpu.VMEM((L,), jnp.int32),
            pltpu.HBM((NS * N,), jnp.float32),
            pltpu.SemaphoreType.DMA,
            pltpu.SemaphoreType.DMA,
            pltpu.SemaphoreType.DMA,
        ],
    )
    def traverse(h_hbm, w_hbm, hd_hbm, tl_hbm, ty_hbm, r_hbm, out_hbm,
                 out_v, hb0, hb1, tb0, tb1, yb0, yb1, wb0, wb1,
                 sh, st, sw, gb, cb, rv, pub,
                 sem_a, sem_b, sem_g):
        hb = (hb0, hb1)
        tb = (tb0, tb1)
        yb = (yb0, yb1)
        wb = (wb0, wb1)
        c = lax.axis_index("c")
        s = lax.axis_index("s")
        b = c * (B // NC) + s // NPART
        part = s % NPART
        ebase = part * e_per_tile
        sems = (sem_a, sem_b)

        iot = lax.broadcasted_iota(jnp.int32, (L,), 0)

        pltpu.sync_copy(r_hbm, rv)
        r_all = rv[...]
        rvec = lax.gather(
            r_all,
            jnp.full((L, 1), b, dtype=jnp.int32),
            lax.GatherDimensionNumbers(
                offset_dims=(), collapsed_slice_dims=(0,),
                start_index_map=(0,)),
            slice_sizes=(1,),
            mode=lax.GatherScatterMode.PROMISE_IN_BOUNDS)
        bofs = jnp.full((L,), b * N, dtype=jnp.int32)

        zf = jnp.zeros((L,), jnp.float32)
        zi = jnp.zeros((L,), jnp.int32)

        def init_out(i, _):
            out_v[pl.ds(i * L, L)] = zf
            return 0
        lax.fori_loop(0, N // L, init_out, 0)

        def init_stage(i, _):
            sh[pl.ds(i * L, L)] = zi
            st[pl.ds(i * L, L)] = zi
            return 0
        lax.fori_loop(0, STG // L, init_stage, 0)

        def start_edges(i, slot):
            off = ebase + i * CHUNK
            pltpu.make_async_copy(hd_hbm.at[pl.ds(off, CHUNK)], hb[slot], sems[slot]).start()
            pltpu.make_async_copy(tl_hbm.at[pl.ds(off, CHUNK)], tb[slot], sems[slot]).start()
            pltpu.make_async_copy(ty_hbm.at[pl.ds(off, CHUNK)], yb[slot], sems[slot]).start()
            pltpu.make_async_copy(w_hbm.at[pl.ds(off, CHUNK)], wb[slot], sems[slot]).start()

        def wait_edges(slot):
            pltpu.make_async_copy(hd_hbm.at[pl.ds(0, CHUNK)], hb[slot], sems[slot]).wait()
            pltpu.make_async_copy(tl_hbm.at[pl.ds(0, CHUNK)], tb[slot], sems[slot]).wait()
            pltpu.make_async_copy(ty_hbm.at[pl.ds(0, CHUNK)], yb[slot], sems[slot]).wait()
            pltpu.make_async_copy(w_hbm.at[pl.ds(0, CHUNK)], wb[slot], sems[slot]).wait()

        def process(slot):
            hbs, tbs, ybs, wbs = hb[slot], tb[slot], yb[slot], wb[slot]

            def scan_body(k, cnt):
                sl = pl.ds(k * L, L)
                m = ybs[sl] == rvec
                pc = plsc.all_reduce_population_count(m)
                if getattr(pc, "ndim", 0):
                    pc = pc[0]

                @pl.when(pc > 0)
                def _():
                    plsc.store_compressed(sh.at[pl.ds(cnt, L)], hbs[sl] + bofs, mask=m)
                    plsc.store_compressed(st.at[pl.ds(cnt, L)], tbs[sl], mask=m)
                    plsc.store_compressed(sw.at[pl.ds(cnt, L)], wbs[sl], mask=m)

                return cnt + pc

            cnt = lax.fori_loop(0, CHUNK // L, scan_body, jnp.int32(0))

            nstream = (cnt + (GW - 1)) // GW

            def fire(k, _):
                pltpu.make_async_copy(
                    h_hbm.at[sh.at[pl.ds(k * GW, GW)]],
                    gb.at[pl.ds(k * GW, GW)], sem_g).start()
                return 0
            lax.fori_loop(0, nstream, fire, 0)

            def drain(k, _):
                pltpu.make_async_copy(
                    h_hbm.at[sh.at[pl.ds(k * GW, GW)]],
                    gb.at[pl.ds(k * GW, GW)], sem_g).wait()
                return 0
            lax.fori_loop(0, nstream, drain, 0)

            cntv = jnp.full((L,), cnt, dtype=jnp.int32)
            nvec = (cnt + (L - 1)) // L

            def smax(k, _):
                sl = pl.ds(k * L, L)
                t = st[sl]
                valid = (iot + k * L) < cntv
                v = jnp.where(valid, sw[sl] * gb[sl], -1.0)
                cur = plsc.load_gather(out_v, [t])

                def unsat(cu):
                    n = plsc.all_reduce_population_count(v > cu)
                    if getattr(n, "ndim", 0):
                        n = n[0]
                    return n > 0

                def wbody(cu):
                    plsc.store_scatter(out_v, [t], v, mask=v > cu)
                    return plsc.load_gather(out_v, [t])

                lax.while_loop(unsat, wbody, cur)
                return 0
            lax.fori_loop(0, nvec, smax, 0)

        start_edges(0, 0)

        def chunk_loop(j, _):
            i1 = 2 * j + 1
            start_edges(i1, 1)
            wait_edges(0)
            process(0)
            start_edges(jnp.minimum(i1 + 1, nchunk - 1), 0)
            wait_edges(1)
            process(1)
            return 0
        lax.fori_loop(0, nchunk // 2, chunk_loop, 0)
        wait_edges(0)

        def absorb(src_row):
            def comb_loop(q, _):
                pltpu.sync_copy(pub.at[pl.ds(src_row * N + q * COMB, COMB)], cb)

                def vmax(k, _):
                    dsl = pl.ds(q * COMB + k * L, L)
                    out_v[dsl] = jnp.maximum(out_v[dsl], cb[pl.ds(k * L, L)])
                    return 0
                lax.fori_loop(0, COMB // L, vmax, 0)
                return 0
            lax.fori_loop(0, N // COMB, comb_loop, 0)

        slot1 = c * (NS // 2) + s // 2

        @pl.when(part % 2 == 1)
        def _():
            pltpu.sync_copy(out_v, pub.at[pl.ds(slot1 * N, N)])
        plsc.subcore_barrier()

        @pl.when(part % 2 == 0)
        def _():
            absorb(slot1)
        plsc.subcore_barrier()

        @pl.when(part == 2)
        def _():
            pltpu.sync_copy(out_v, pub.at[pl.ds(slot1 * N, N)])
        plsc.subcore_barrier()

        @pl.when(part == 0)
        def _():
            absorb(slot1 + 1)
            pltpu.sync_copy(out_v, out_hbm.at[pl.ds(b * N, N)])

    return traverse


def kernel(h_prob, edge_weight, edge_index, edge_type, r_index):
    B, N = h_prob.shape
    E = edge_type.shape[0]
    heads = edge_index[0]
    tails = edge_index[1]
    hflat = h_prob.reshape(-1)
    r_pad = jnp.zeros((L,), jnp.int32).at[:B].set(r_index)
    fn = _build(B, N, E)
    out = fn(hflat, edge_weight, heads, tails, edge_type, r_pad)
    return out.reshape(B, N)
